# Initial kernel scaffold; baseline (speedup 1.0000x reference)
import jax
import jax.numpy as jnp
from jax.experimental import pallas as pl


def kernel(x, table):
    raise NotImplementedError("write your pallas kernel here")


# speedup vs baseline: 2.9674x; 2.9674x over previous
import functools

import jax
import jax.numpy as jnp
from jax import lax
from jax.experimental import pallas as pl
from jax.experimental.pallas import tpu as pltpu, tpu_sc as plsc

_VOCAB = 100000
_DIM = 128
_BATCH = 4096
_SEQ = 50
_B = _BATCH * _SEQ

_NC = 2
_NS = 16
_NW = _NC * _NS
_B_PER_W = _B // _NW
_CHUNK = 128
_NCHUNK = _B_PER_W // _CHUNK


def _body(x_hbm, table_hbm, out_hbm, idx_v, rows_v, sem):
    wid = lax.axis_index("s") * _NC + lax.axis_index("c")
    base = wid * _B_PER_W
    pltpu.sync_copy(x_hbm.at[pl.ds(base, _B_PER_W)], idx_v)

    def step(j, carry):
        off = j * _CHUNK
        pltpu.async_copy(
            table_hbm.at[idx_v.at[pl.ds(off, _CHUNK)]], rows_v, sem
        ).wait()
        pltpu.sync_copy(rows_v, out_hbm.at[pl.ds(base + off, _CHUNK)])
        return carry

    lax.fori_loop(0, _NCHUNK, step, 0)


@jax.jit
def _embed(x_flat, table):
    mesh = plsc.VectorSubcoreMesh(core_axis_name="c", subcore_axis_name="s")
    f = pl.kernel(
        _body,
        out_type=jax.ShapeDtypeStruct((_B, _DIM), jnp.float32),
        mesh=mesh,
        scratch_types=[
            pltpu.VMEM((_B_PER_W,), jnp.int32),
            pltpu.VMEM((_CHUNK, _DIM), jnp.float32),
            pltpu.SemaphoreType.DMA,
        ],
    )
    return f(x_flat, table)


def kernel(x, table):
    x_flat = x.reshape(-1).astype(jnp.int32)
    out = _embed(x_flat, table)
    return out.reshape(_BATCH, _SEQ, _DIM)


# speedup vs baseline: 3.3498x; 1.1289x over previous
import jax
import jax.numpy as jnp
from jax import lax
from jax.experimental import pallas as pl
from jax.experimental.pallas import tpu as pltpu, tpu_sc as plsc

_VOCAB = 100000
_DIM = 128
_BATCH = 4096
_SEQ = 50
_B = _BATCH * _SEQ

_NC = 2
_NS = 16
_NW = _NC * _NS
_B_PER_W = _B // _NW
_CHUNK = 128
_NCHUNK = _B_PER_W // _CHUNK
_NBUF = 5
_NROUND = _NCHUNK // _NBUF


def _body(x_hbm, table_hbm, out_hbm, idx_v, rows_v, gsem, ssem):
    wid = lax.axis_index("s") * _NC + lax.axis_index("c")
    base = wid * _B_PER_W
    pltpu.sync_copy(x_hbm.at[pl.ds(base, _B_PER_W)], idx_v)

    def gather_start(j, b):
        pltpu.async_copy(
            table_hbm.at[idx_v.at[pl.ds(j * _CHUNK, _CHUNK)]],
            rows_v.at[b],
            gsem.at[b],
        )

    def gather_wait(b):
        pltpu.make_async_copy(
            table_hbm.at[idx_v.at[pl.ds(0, _CHUNK)]], rows_v.at[b], gsem.at[b]
        ).wait()

    def store_start(j, b):
        pltpu.async_copy(
            rows_v.at[b], out_hbm.at[pl.ds(base + j * _CHUNK, _CHUNK)], ssem.at[b]
        )

    def store_wait(b):
        pltpu.make_async_copy(
            rows_v.at[b], out_hbm.at[pl.ds(base, _CHUNK)], ssem.at[b]
        ).wait()

    for b in range(_NBUF):
        gather_start(b, b)

    def round_fn(g, carry):
        for b in range(_NBUF):
            j = g * _NBUF + b
            gather_wait(b)
            store_start(j, b)
            store_wait(b)
            gather_start(j + _NBUF, b)
        return carry

    lax.fori_loop(0, _NROUND - 1, round_fn, 0)

    for b in range(_NBUF):
        j = (_NROUND - 1) * _NBUF + b
        gather_wait(b)
        store_start(j, b)
    for b in range(_NBUF):
        store_wait(b)


@jax.jit
def _embed(x_flat, table):
    mesh = plsc.VectorSubcoreMesh(core_axis_name="c", subcore_axis_name="s")
    f = pl.kernel(
        _body,
        out_type=jax.ShapeDtypeStruct((_B, _DIM), jnp.float32),
        mesh=mesh,
        scratch_types=[
            pltpu.VMEM((_B_PER_W,), jnp.int32),
            pltpu.VMEM((_NBUF, _CHUNK, _DIM), jnp.float32),
            pltpu.SemaphoreType.DMA((_NBUF,)),
            pltpu.SemaphoreType.DMA((_NBUF,)),
        ],
    )
    return f(x_flat, table)


def kernel(x, table):
    x_flat = x.reshape(-1).astype(jnp.int32)
    out = _embed(x_flat, table)
    return out.reshape(_BATCH, _SEQ, _DIM)


# speedup vs baseline: 5.9148x; 1.7657x over previous
import jax
import jax.numpy as jnp
from jax import lax
from jax.experimental import pallas as pl
from jax.experimental.pallas import tpu as pltpu, tpu_sc as plsc

_VOCAB = 100000
_DIM = 128
_BATCH = 4096
_SEQ = 50
_SEQ_PAD = 56

_NC = 2
_NS = 16
_NW = _NC * _NS
_BT_PER_W = _BATCH // _NW
_IDX_PER_W = _BT_PER_W * _SEQ_PAD
_NBUF = 4
_NROUND = _BT_PER_W // _NBUF


def _body(xp_hbm, table_hbm, out_hbm, idx_v, rows_v, gsem, ssem):
    wid = lax.axis_index("s") * _NC + lax.axis_index("c")
    base_b = wid * _BT_PER_W
    pltpu.sync_copy(xp_hbm.at[pl.ds(wid * _IDX_PER_W, _IDX_PER_W)], idx_v)

    def gather_start(j, b):
        pltpu.async_copy(
            table_hbm.at[idx_v.at[pl.ds(j * _SEQ_PAD, _SEQ)]],
            rows_v.at[b],
            gsem.at[b],
        )

    def gather_wait(b):
        pltpu.make_async_copy(
            table_hbm.at[idx_v.at[pl.ds(0, _SEQ)]], rows_v.at[b], gsem.at[b]
        ).wait()

    def store_start(j, b):
        pltpu.async_copy(
            rows_v.at[b], out_hbm.at[base_b + j], ssem.at[b]
        )

    def store_wait(b):
        pltpu.make_async_copy(
            rows_v.at[b], out_hbm.at[base_b], ssem.at[b]
        ).wait()

    for b in range(_NBUF):
        gather_start(b, b)

    def round_fn(g, carry):
        for b in range(_NBUF):
            j = g * _NBUF + b
            gather_wait(b)
            store_start(j, b)
            store_wait(b)
            gather_start(j + _NBUF, b)
        return carry

    lax.fori_loop(0, _NROUND - 1, round_fn, 0)

    for b in range(_NBUF):
        j = (_NROUND - 1) * _NBUF + b
        gather_wait(b)
        store_start(j, b)
    for b in range(_NBUF):
        store_wait(b)


@jax.jit
def _embed(xp_flat, table):
    mesh = plsc.VectorSubcoreMesh(core_axis_name="c", subcore_axis_name="s")
    f = pl.kernel(
        _body,
        out_type=jax.ShapeDtypeStruct((_BATCH, _SEQ, _DIM), jnp.float32),
        mesh=mesh,
        scratch_types=[
            pltpu.VMEM((_IDX_PER_W,), jnp.int32),
            pltpu.VMEM((_NBUF, _SEQ, _DIM), jnp.float32),
            pltpu.SemaphoreType.DMA((_NBUF,)),
            pltpu.SemaphoreType.DMA((_NBUF,)),
        ],
        compiler_params=pltpu.CompilerParams(use_tc_tiling_on_sc=True),
    )
    return f(xp_flat, table)


def kernel(x, table):
    xp = jnp.pad(x.astype(jnp.int32), ((0, 0), (0, _SEQ_PAD - _SEQ)))
    return _embed(xp.reshape(-1), table)


# speedup vs baseline: 5.9209x; 1.0010x over previous
import jax
import jax.numpy as jnp
from jax import lax
from jax.experimental import pallas as pl
from jax.experimental.pallas import tpu as pltpu, tpu_sc as plsc

_VOCAB = 100000
_DIM = 128
_BATCH = 4096
_SEQ = 50

_NC = 2
_NS = 16
_NW = _NC * _NS
_BT_PER_W = _BATCH // _NW
_K = 4
_CHUNK = _K * _SEQ
_NCHUNK = _BT_PER_W // _K
_IDX_PER_W = _BT_PER_W * _SEQ
_NBUF = 4


def _body(xp_hbm, table_hbm, out_hbm, idx_v, rows_v, gsem, ssem):
    wid = lax.axis_index("s") * _NC + lax.axis_index("c")
    base_b = wid * _BT_PER_W
    pltpu.sync_copy(xp_hbm.at[pl.ds(wid * _IDX_PER_W, _IDX_PER_W)], idx_v)

    def gather_start(j, b):
        pltpu.async_copy(
            table_hbm.at[idx_v.at[pl.ds(j * _CHUNK, _CHUNK)]],
            rows_v.at[b],
            gsem.at[b],
        )

    def gather_wait(b):
        pltpu.make_async_copy(
            table_hbm.at[idx_v.at[pl.ds(0, _CHUNK)]], rows_v.at[b], gsem.at[b]
        ).wait()

    def store_start(j, b):
        for i in range(_K):
            pltpu.async_copy(
                rows_v.at[b, pl.ds(i * _SEQ, _SEQ)],
                out_hbm.at[base_b + j * _K + i],
                ssem.at[b],
            )

    def store_wait(b):
        for _ in range(_K):
            pltpu.make_async_copy(
                rows_v.at[b, pl.ds(0, _SEQ)], out_hbm.at[base_b], ssem.at[b]
            ).wait()

    for b in range(_NBUF):
        gather_start(b, b)

    def round_fn(g, carry):
        for b in range(_NBUF):
            j = g * _NBUF + b
            gather_wait(b)
            store_start(j, b)
            store_wait(b)
            gather_start(j + _NBUF, b)
        return carry

    lax.fori_loop(0, _NCHUNK // _NBUF - 1, round_fn, 0)

    for b in range(_NBUF):
        j = (_NCHUNK // _NBUF - 1) * _NBUF + b
        gather_wait(b)
        store_start(j, b)
    for b in range(_NBUF):
        store_wait(b)


@jax.jit
def _embed(xp_flat, table):
    mesh = plsc.VectorSubcoreMesh(core_axis_name="c", subcore_axis_name="s")
    f = pl.kernel(
        _body,
        out_type=jax.ShapeDtypeStruct((_BATCH, _SEQ, _DIM), jnp.float32),
        mesh=mesh,
        scratch_types=[
            pltpu.VMEM((_IDX_PER_W,), jnp.int32),
            pltpu.VMEM((_NBUF, _CHUNK, _DIM), jnp.float32),
            pltpu.SemaphoreType.DMA((_NBUF,)),
            pltpu.SemaphoreType.DMA((_NBUF,)),
        ],
        compiler_params=pltpu.CompilerParams(use_tc_tiling_on_sc=True),
    )
    return f(xp_flat, table)


def kernel(x, table):
    return _embed(x.astype(jnp.int32).reshape(-1), table)


# speedup vs baseline: 5.9268x; 1.0010x over previous
import jax
import jax.numpy as jnp
from jax import lax
from jax.experimental import pallas as pl
from jax.experimental.pallas import tpu as pltpu, tpu_sc as plsc

_VOCAB = 100000
_DIM = 128
_BATCH = 4096
_SEQ = 50

_NC = 2
_NS = 16
_NW = _NC * _NS
_BT_PER_W = _BATCH // _NW
_K = 4
_CHUNK = _K * _SEQ
_NCHUNK = _BT_PER_W // _K
_IDX_PER_W = _BT_PER_W * _SEQ
_NBUF = 4


def _body(xp_hbm, table_hbm, out_hbm, idx_v, rows_v, gsem, ssem):
    wid = lax.axis_index("s") * _NC + lax.axis_index("c")
    base_b = wid * _BT_PER_W
    pltpu.sync_copy(xp_hbm.at[pl.ds(wid * _IDX_PER_W, _IDX_PER_W)], idx_v)

    def gather_start(j, b):
        pltpu.async_copy(
            table_hbm.at[idx_v.at[pl.ds(j * _CHUNK, _CHUNK)]],
            rows_v.at[b],
            gsem.at[b],
        )

    def gather_wait(b):
        pltpu.make_async_copy(
            table_hbm.at[idx_v.at[pl.ds(0, _CHUNK)]], rows_v.at[b], gsem.at[b]
        ).wait()

    def store_start(j, b):
        for i in range(_K):
            pltpu.async_copy(
                rows_v.at[b, pl.ds(i * _SEQ, _SEQ)],
                out_hbm.at[base_b + j * _K + i],
                ssem.at[b],
            )

    def store_wait(b):
        for _ in range(_K):
            pltpu.make_async_copy(
                rows_v.at[b, pl.ds(0, _SEQ)], out_hbm.at[base_b], ssem.at[b]
            ).wait()

    for b in range(_NBUF):
        gather_start(b, b)

    n_rounds = _NCHUNK // _NBUF

    for b in range(_NBUF):
        j = b
        gather_wait(b)
        store_start(j, b)
        if j >= 1:
            pb = b - 1
            store_wait(pb)
            gather_start(j - 1 + _NBUF, pb)

    def round_fn(g, carry):
        for b in range(_NBUF):
            j = g * _NBUF + b
            gather_wait(b)
            store_start(j, b)
            pb = (b - 1) % _NBUF
            store_wait(pb)
            gather_start(j - 1 + _NBUF, pb)
        return carry

    lax.fori_loop(1, n_rounds - 1, round_fn, 0)

    for b in range(_NBUF):
        j = (n_rounds - 1) * _NBUF + b
        gather_wait(b)
        store_start(j, b)
        pb = (b - 1) % _NBUF
        store_wait(pb)
        if j - 1 + _NBUF <= _NCHUNK - 1:
            gather_start(j - 1 + _NBUF, pb)
    store_wait(_NBUF - 1)


@jax.jit
def _embed(xp_flat, table):
    mesh = plsc.VectorSubcoreMesh(core_axis_name="c", subcore_axis_name="s")
    f = pl.kernel(
        _body,
        out_type=jax.ShapeDtypeStruct((_BATCH, _SEQ, _DIM), jnp.float32),
        mesh=mesh,
        scratch_types=[
            pltpu.VMEM((_IDX_PER_W,), jnp.int32),
            pltpu.VMEM((_NBUF, _CHUNK, _DIM), jnp.float32),
            pltpu.SemaphoreType.DMA((_NBUF,)),
            pltpu.SemaphoreType.DMA((_NBUF,)),
        ],
        compiler_params=pltpu.CompilerParams(use_tc_tiling_on_sc=True),
    )
    return f(xp_flat, table)


def kernel(x, table):
    return _embed(x.astype(jnp.int32).reshape(-1), table)


# speedup vs baseline: 5.9425x; 1.0026x over previous
import jax
import jax.numpy as jnp
from jax import lax
from jax.experimental import pallas as pl
from jax.experimental.pallas import tpu as pltpu, tpu_sc as plsc

_VOCAB = 100000
_DIM = 128
_BATCH = 4096
_SEQ = 50
_SEQ_PAD = 56

_NC = 2
_NS = 16
_NW = _NC * _NS
_BT_PER_W = _BATCH // _NW
_IDX_PER_W = _BT_PER_W * _SEQ_PAD
_NBUF = 8


def _body(xp_hbm, table_hbm, out_hbm, idx_v, rows_v, gsem, ssem):
    wid = lax.axis_index("s") * _NC + lax.axis_index("c")
    base_b = wid * _BT_PER_W
    pltpu.sync_copy(xp_hbm.at[pl.ds(wid * _IDX_PER_W, _IDX_PER_W)], idx_v)

    def gather_start(j, b):
        pltpu.async_copy(
            table_hbm.at[idx_v.at[pl.ds(j * _SEQ_PAD, _SEQ)]],
            rows_v.at[b],
            gsem.at[b],
        )

    def gather_wait(b):
        pltpu.make_async_copy(
            table_hbm.at[idx_v.at[pl.ds(0, _SEQ)]], rows_v.at[b], gsem.at[b]
        ).wait()

    def store_start(j, b):
        pltpu.async_copy(rows_v.at[b], out_hbm.at[base_b + j], ssem.at[b])

    def store_wait(b):
        pltpu.make_async_copy(
            rows_v.at[b], out_hbm.at[base_b], ssem.at[b]
        ).wait()

    for b in range(_NBUF):
        gather_start(b, b)

    n_rounds = _BT_PER_W // _NBUF

    for b in range(_NBUF):
        j = b
        gather_wait(b)
        store_start(j, b)
        if j >= 1:
            pb = b - 1
            store_wait(pb)
            gather_start(j - 1 + _NBUF, pb)

    def round_fn(g, carry):
        for b in range(_NBUF):
            j = g * _NBUF + b
            gather_wait(b)
            store_start(j, b)
            pb = (b - 1) % _NBUF
            store_wait(pb)
            gather_start(j - 1 + _NBUF, pb)
        return carry

    lax.fori_loop(1, n_rounds - 1, round_fn, 0)

    for b in range(_NBUF):
        j = (n_rounds - 1) * _NBUF + b
        gather_wait(b)
        store_start(j, b)
        pb = (b - 1) % _NBUF
        store_wait(pb)
        if j - 1 + _NBUF <= _BT_PER_W - 1:
            gather_start(j - 1 + _NBUF, pb)
    store_wait(_NBUF - 1)


@jax.jit
def _embed(xp_flat, table):
    mesh = plsc.VectorSubcoreMesh(core_axis_name="c", subcore_axis_name="s")
    f = pl.kernel(
        _body,
        out_type=jax.ShapeDtypeStruct((_BATCH, _SEQ, _DIM), jnp.float32),
        mesh=mesh,
        scratch_types=[
            pltpu.VMEM((_IDX_PER_W,), jnp.int32),
            pltpu.VMEM((_NBUF, _SEQ, _DIM), jnp.float32),
            pltpu.SemaphoreType.DMA((_NBUF,)),
            pltpu.SemaphoreType.DMA((_NBUF,)),
        ],
        compiler_params=pltpu.CompilerParams(use_tc_tiling_on_sc=True),
    )
    return f(xp_flat, table)


def kernel(x, table):
    xp = jnp.pad(x.astype(jnp.int32), ((0, 0), (0, _SEQ_PAD - _SEQ)))
    return _embed(xp.reshape(-1), table)
